# baseline (speedup 1.0000x reference)
import functools

import jax
import jax.numpy as jnp
from jax import lax
from jax.experimental import pallas as pl
from jax.experimental.pallas import tpu as pltpu
from jax.experimental.pallas import tpu_sc as plsc

NC = 2
NS = 16
NW = NC * NS
LANES = 16

R = 8


@functools.lru_cache(maxsize=None)
def _build_sc(k_rows, rows, d):
    per_w = k_rows // NW
    n_chunks = per_w // R
    assert per_w % R == 0 and n_chunks % 2 == 0
    dh = d // 2

    mesh = plsc.VectorSubcoreMesh(
        core_axis_name="c", subcore_axis_name="s", num_cores=NC, num_subcores=NS
    )

    @functools.partial(
        pl.kernel,
        out_type=jax.ShapeDtypeStruct((k_rows, d), jnp.float32),
        mesh=mesh,
        scratch_types=[
            pltpu.VMEM((LANES,), jnp.float32),
            [pltpu.VMEM((R, d), jnp.float32) for _ in range(6)],
            [pltpu.VMEM((R, dh), jnp.float32) for _ in range(2)],
            [pltpu.SemaphoreType.DMA for _ in range(4)],
        ],
    )
    def run(t_hbm, x_hbm, x1_hbm, m_hbm, out_hbm, tb, bufs, obufs, sems):
        xb = (bufs[0], bufs[3])
        x1b = (bufs[1], bufs[4])
        mb = (bufs[2], bufs[5])
        in_sem = (sems[0], sems[1])
        out_sem = (sems[2], sems[3])

        wid = lax.axis_index("s") * NC + lax.axis_index("c")
        base = wid * per_w

        pltpu.sync_copy(t_hbm, tb)
        tv = tb[...]

        def start_in(c, b):
            row = base + c * R
            pltpu.async_copy(x_hbm.at[pl.ds(row, R), :], xb[b], in_sem[b])
            pltpu.async_copy(x1_hbm.at[pl.ds(row, R), :], x1b[b], in_sem[b])
            pltpu.async_copy(m_hbm.at[pl.ds(row, R), :], mb[b], in_sem[b])

        def wait_in(c, b):
            row = base + c * R
            pltpu.make_async_copy(x_hbm.at[pl.ds(row, R), :], xb[b], in_sem[b]).wait()
            pltpu.make_async_copy(x_hbm.at[pl.ds(row, R), :], x1b[b], in_sem[b]).wait()
            pltpu.make_async_copy(x_hbm.at[pl.ds(row, R), :], mb[b], in_sem[b]).wait()

        start_in(0, 0)
        start_in(1, 1)

        @pl.loop(0, n_chunks, step=2)
        def _blocks(i):
            for b in range(2):
                c = i + b
                row = base + c * R
                wait_in(c, b)
                xr, x1r, mr = xb[b], x1b[b], mb[b]

                for h in range(2):
                    ob = obufs[h]

                    if b == 0:
                        @pl.when(i > 0)
                        def _():
                            pltpu.make_async_copy(
                                ob,
                                out_hbm.at[pl.ds(row, R), pl.ds(h * dh, dh)],
                                out_sem[h],
                            ).wait()
                    else:
                        pltpu.make_async_copy(
                            ob,
                            out_hbm.at[pl.ds(row, R), pl.ds(h * dh, dh)],
                            out_sem[h],
                        ).wait()

                    for r in range(R):
                        @plsc.parallel_loop(0, dh, step=LANES, unroll=8)
                        def _compute(j):
                            jh = j + h * dh
                            mv = mr[r, pl.ds(jh, LANES)]
                            ob[r, pl.ds(j, LANES)] = jnp.where(
                                mv >= tv,
                                xr[r, pl.ds(jh, LANES)],
                                x1r[r, pl.ds(jh, LANES)],
                            )

                    pltpu.async_copy(
                        ob, out_hbm.at[pl.ds(row, R), pl.ds(h * dh, dh)], out_sem[h]
                    )

                @pl.when(c + 2 < n_chunks)
                def _():
                    start_in(c + 2, b)

        row_last = base + (n_chunks - 1) * R
        for h in range(2):
            pltpu.make_async_copy(
                obufs[h],
                out_hbm.at[pl.ds(row_last, R), pl.ds(h * dh, dh)],
                out_sem[h],
            ).wait()

    return run


def _tc_body(t_ref, x_ref, x1_ref, m_ref, o_ref):
    t = t_ref[0]
    o_ref[...] = jnp.where(m_ref[...] >= t, x_ref[...], x1_ref[...])


def kernel(x, x1, mask, threshold):
    B, S, D = x.shape
    rows = B * S
    K = rows // 2
    x2 = x.reshape(rows, D)
    x12 = x1.reshape(rows, D)
    m2 = mask.reshape(rows, D)
    t16 = jnp.broadcast_to(threshold.astype(jnp.float32), (LANES,))

    sc_out = _build_sc(K, rows, D)(t16, x2, x12, m2)

    ROWS = 512
    tc_rows = rows - K
    off_blocks = K // ROWS
    tc_out = pl.pallas_call(
        _tc_body,
        grid=(tc_rows // ROWS,),
        in_specs=[
            pl.BlockSpec(memory_space=pltpu.SMEM),
            pl.BlockSpec((ROWS, D), lambda i: (i + off_blocks, 0)),
            pl.BlockSpec((ROWS, D), lambda i: (i + off_blocks, 0)),
            pl.BlockSpec((ROWS, D), lambda i: (i + off_blocks, 0)),
        ],
        out_specs=pl.BlockSpec((ROWS, D), lambda i: (i, 0)),
        out_shape=jax.ShapeDtypeStruct((tc_rows, D), jnp.float32),
    )(threshold.reshape(1), x2, x12, m2)

    out = jnp.concatenate([sc_out, tc_out], axis=0)
    return out.reshape(B, S, D)


# speedup vs baseline: 1.3042x; 1.3042x over previous
import functools

import jax
import jax.numpy as jnp
from jax import lax
from jax.experimental import pallas as pl
from jax.experimental.pallas import tpu as pltpu
from jax.experimental.pallas import tpu_sc as plsc

NC = 2
NS = 16
NW = NC * NS
LANES = 16

R = 2
NBUF = 4


@functools.lru_cache(maxsize=None)
def _build(rows, d):
    per_w = rows // NW
    n_chunks = per_w // R
    assert per_w % R == 0 and n_chunks % NBUF == 0
    dh = d // 2

    mesh = plsc.VectorSubcoreMesh(
        core_axis_name="c", subcore_axis_name="s", num_cores=NC, num_subcores=NS
    )

    @functools.partial(
        pl.kernel,
        out_type=jax.ShapeDtypeStruct((rows, d), jnp.float32),
        mesh=mesh,
        scratch_types=[
            pltpu.VMEM((LANES,), jnp.float32),
            [pltpu.VMEM((R, d), jnp.float32) for _ in range(12)],
            [pltpu.VMEM((R, d), jnp.float32) for _ in range(4)],
            [pltpu.SemaphoreType.DMA for _ in range(8)],
        ],
    )
    def run(t_hbm, x_hbm, x1_hbm, m_hbm, out_hbm, tb, bufs, obufs, sems):
        xb = (bufs[0], bufs[3], bufs[6], bufs[9])
        x1b = (bufs[1], bufs[4], bufs[7], bufs[10])
        mb = (bufs[2], bufs[5], bufs[8], bufs[11])
        in_sem = (sems[0], sems[1], sems[2], sems[3])
        out_sem = (sems[4], sems[5], sems[6], sems[7])

        wid = lax.axis_index("s") * NC + lax.axis_index("c")
        base = wid * per_w

        pltpu.sync_copy(t_hbm, tb)
        tv = tb[...]

        def start_in(c, b):
            row = base + c * R
            pltpu.async_copy(x_hbm.at[pl.ds(row, R), :], xb[b], in_sem[b])
            pltpu.async_copy(x1_hbm.at[pl.ds(row, R), :], x1b[b], in_sem[b])
            pltpu.async_copy(m_hbm.at[pl.ds(row, R), :], mb[b], in_sem[b])

        def wait_in(c, b):
            row = base + c * R
            pltpu.make_async_copy(x_hbm.at[pl.ds(row, R), :], xb[b], in_sem[b]).wait()
            pltpu.make_async_copy(x_hbm.at[pl.ds(row, R), :], x1b[b], in_sem[b]).wait()
            pltpu.make_async_copy(x_hbm.at[pl.ds(row, R), :], mb[b], in_sem[b]).wait()

        start_in(0, 0)
        start_in(1, 1)
        start_in(2, 2)
        start_in(3, 3)

        @pl.loop(0, n_chunks, step=NBUF)
        def _blocks(i):
            for b in range(NBUF):
                c = i + b
                row = base + c * R
                wait_in(c, b)
                xr, x1r, mr = xb[b], x1b[b], mb[b]

                ob = obufs[b]

                @pl.when(i > 0)
                def _():
                    pltpu.make_async_copy(
                        ob, out_hbm.at[pl.ds(row, R), :], out_sem[b]
                    ).wait()

                for r in range(R):
                    @plsc.parallel_loop(0, d, step=LANES, unroll=8)
                    def _compute(j):
                        mv = mr[r, pl.ds(j, LANES)]
                        ob[r, pl.ds(j, LANES)] = jnp.where(
                            mv >= tv,
                            xr[r, pl.ds(j, LANES)],
                            x1r[r, pl.ds(j, LANES)],
                        )

                pltpu.async_copy(ob, out_hbm.at[pl.ds(row, R), :], out_sem[b])

                @pl.when(c + NBUF < n_chunks)
                def _():
                    start_in(c + NBUF, b)

        row_last = base + (n_chunks - 1) * R
        for b in range(NBUF):
            pltpu.make_async_copy(
                obufs[b], out_hbm.at[pl.ds(row_last, R), :], out_sem[b]
            ).wait()

    return run


def kernel(x, x1, mask, threshold):
    B, S, D = x.shape
    rows = B * S
    t16 = jnp.broadcast_to(threshold.astype(jnp.float32), (LANES,))
    run = _build(rows, D)
    out = run(t16, x.reshape(rows, D), x1.reshape(rows, D), mask.reshape(rows, D))
    return out.reshape(B, S, D)
